# baseline (speedup 1.0000x reference)
import functools

import jax
import jax.numpy as jnp
from jax import lax
from jax.experimental import pallas as pl
from jax.experimental.pallas import tpu as pltpu
from jax.experimental.pallas import tpu_sc as plsc

N = 10000
E = 320000
D = 128
G = 16
NC, NS = 2, 16
NW = NC * NS
CH = 128
GRP = 8
NG0 = 16
NG1 = 4
CPW0 = GRP * NG0
CPW1 = GRP * NG1
E0 = NS * CPW0 * CH
E1 = NS * CPW1 * CH
NPAD = N + 112
RPT = NPAD // NS

F32 = jnp.float32
HI = lax.Precision.HIGHEST

_mesh = plsc.VectorSubcoreMesh(
    core_axis_name="c", subcore_axis_name="s", num_cores=NC, num_subcores=NS
)


def _make_segsum_body(with_deg):
    def body_fn(*args):
        if with_deg:
            (y, srcw, dstw, zacc, zdeg, sacc_out, deg_out, accum,
             src_v0, src_v1, dst_v0, dst_v1, rows_v0, rows_v1, deg_v,
             sm0, sm1, sm2, sm3, gs0, gs1, ss0, ss1) = args
        else:
            (y, srcw, dstw, zacc, sacc_out, accum,
             src_v0, src_v1, dst_v0, dst_v1, rows_v0, rows_v1,
             sm0, sm1, sm2, sm3, gs0, gs1, ss0, ss1) = args
        c = lax.axis_index("c")
        s = lax.axis_index("s")
        r0 = s * RPT
        pltpu.sync_copy(zacc.at[pl.ds(r0, RPT)], accum.at[pl.ds(r0, RPT)])
        if with_deg:
            pltpu.sync_copy(zdeg, deg_v)
            ones16 = jnp.ones((16,), F32)
        w = c * NS + s
        sidx = [src_v0, src_v1]
        didx = [dst_v0, dst_v1]
        rows = [rows_v0, rows_v1]
        ssem = [sm0, sm1]
        dsem = [sm2, sm3]
        gsem = [gs0, gs1]
        csem = [ss0, ss1]
        plsc.subcore_barrier()

        def run_chunks(t_lo, t_hi, ng_hi):
            g_lo = t_lo // GRP
            slab = [[None, None], [None, None]]
            p_lo = g_lo % 2
            slab[p_lo][0] = pltpu.async_copy(
                srcw.at[w, pl.ds(g_lo * GRP, GRP)], sidx[p_lo], ssem[p_lo])
            slab[p_lo][1] = pltpu.async_copy(
                dstw.at[w, pl.ds(g_lo * GRP, GRP)], didx[p_lo], dsem[p_lo])
            sca = [None, None]
            prev = None

            def flush_prev(prev):
                pb, pdesc, pp, pr = prev
                pdesc.wait()
                sca[pb] = pltpu.async_copy(
                    rows[pb], accum.at[didx[pp].at[pr]], csem[pb], add=True)
                if with_deg:
                    for k in range(CH // 16):
                        idx = didx[pp][pr, pl.ds(k * 16, 16)]
                        plsc.addupdate_scatter(deg_v, [idx], ones16)

            for t in range(t_lo, t_hi):
                b = t % 2
                g = t // GRP
                p = g % 2
                r = t - g * GRP
                if r == 0:
                    slab[p][0].wait()
                    slab[p][1].wait()
                if sca[b] is not None:
                    sca[b].wait()
                    sca[b] = None
                gat = pltpu.async_copy(y.at[sidx[p].at[r]], rows[b], gsem[b])
                if r == 1 and g + 1 < ng_hi:
                    q = 1 - p
                    slab[q][0] = pltpu.async_copy(
                        srcw.at[w, pl.ds((g + 1) * GRP, GRP)], sidx[q], ssem[q])
                    slab[q][1] = pltpu.async_copy(
                        dstw.at[w, pl.ds((g + 1) * GRP, GRP)], didx[q], dsem[q])
                if prev is not None:
                    flush_prev(prev)
                prev = (b, gat, p, r)
            flush_prev(prev)
            for d in sca:
                if d is not None:
                    d.wait()

        run_chunks(0, CPW1, NG1)

        @pl.when(c == 0)
        def _():
            run_chunks(CPW1, CPW0, NG0)

        if with_deg:
            pltpu.sync_copy(deg_v, deg_out.at[w])
        plsc.subcore_barrier()
        pltpu.sync_copy(accum.at[pl.ds(r0, RPT)], sacc_out.at[c, pl.ds(r0, RPT)])

    return body_fn


_SEMS = [pltpu.SemaphoreType.DMA] * 8

_segsum_deg = functools.partial(
    pl.kernel,
    out_type=(
        jax.ShapeDtypeStruct((NC, NPAD, D), F32),
        jax.ShapeDtypeStruct((NW, NPAD), F32),
    ),
    mesh=_mesh,
    compiler_params=pltpu.CompilerParams(needs_layout_passes=False),
    scratch_types=[
        pltpu.VMEM_SHARED((NPAD, D), F32),
        pltpu.VMEM((GRP, CH), jnp.int32),
        pltpu.VMEM((GRP, CH), jnp.int32),
        pltpu.VMEM((GRP, CH), jnp.int32),
        pltpu.VMEM((GRP, CH), jnp.int32),
        pltpu.VMEM((CH, D), F32),
        pltpu.VMEM((CH, D), F32),
        pltpu.VMEM((NPAD,), F32),
    ] + _SEMS,
)(_make_segsum_body(True))


_segsum = functools.partial(
    pl.kernel,
    out_type=jax.ShapeDtypeStruct((NC, NPAD, D), F32),
    mesh=_mesh,
    compiler_params=pltpu.CompilerParams(needs_layout_passes=False),
    scratch_types=[
        pltpu.VMEM_SHARED((NPAD, D), F32),
        pltpu.VMEM((GRP, CH), jnp.int32),
        pltpu.VMEM((GRP, CH), jnp.int32),
        pltpu.VMEM((GRP, CH), jnp.int32),
        pltpu.VMEM((GRP, CH), jnp.int32),
        pltpu.VMEM((CH, D), F32),
        pltpu.VMEM((CH, D), F32),
    ] + _SEMS,
)(_make_segsum_body(False))


def _tc_pre_body(x_ref, wl_ref, wr_ref, b_ref, y_ref, p_ref):
    xv = x_ref[...]
    y_ref[...] = jnp.dot(xv, wl_ref[...], preferred_element_type=F32,
                         precision=HI)
    p_ref[...] = jnp.dot(xv, wr_ref[...], preferred_element_type=F32,
                         precision=HI) + b_ref[...]


def _tc_pre(x, wl, wr, b):
    return pl.pallas_call(
        _tc_pre_body,
        out_shape=(jax.ShapeDtypeStruct((N, D), F32),
                   jax.ShapeDtypeStruct((N, D), F32)),
    )(x, wl, wr, b)


def _tc_mid_body(s0_ref, s1_ref, dt_ref, p1_ref, wl_ref, wr_ref,
                 b_ref, y_ref, p_ref):
    deg = jnp.maximum(jnp.sum(dt_ref[...], axis=1, keepdims=True), 1.0)
    h = jax.nn.relu((s0_ref[...] + s1_ref[...]) / deg + p1_ref[...])
    y_ref[...] = jnp.dot(h, wl_ref[...], preferred_element_type=F32,
                         precision=HI)
    p_ref[...] = jnp.dot(h, wr_ref[...], preferred_element_type=F32,
                         precision=HI) + b_ref[...]


def _tc_mid(s0, s1, dt, p1, wl, wr, b):
    return pl.pallas_call(
        _tc_mid_body,
        out_shape=(jax.ShapeDtypeStruct((N, D), F32),
                   jax.ShapeDtypeStruct((N, D), F32)),
    )(s0, s1, dt, p1, wl, wr, b)


def _tc_final_body(s0_ref, s1_ref, dt_ref, p2_ref, batch_ref,
                   out_ref):
    deg = jnp.maximum(jnp.sum(dt_ref[...], axis=1, keepdims=True), 1.0)
    h = (s0_ref[...] + s1_ref[...]) / deg + p2_ref[...]
    gids = lax.broadcasted_iota(jnp.int32, (G, N), 0)
    onehot = (gids == batch_ref[...]).astype(F32)
    sums = jnp.dot(onehot, h, preferred_element_type=F32, precision=HI)
    counts = jnp.sum(onehot, axis=1, keepdims=True)
    out_ref[...] = sums / jnp.maximum(counts, 1.0)


def _tc_final(s0, s1, dt, p2, batch_row):
    return pl.pallas_call(
        _tc_final_body,
        out_shape=jax.ShapeDtypeStruct((G, D), F32),
    )(s0, s1, dt, p2, batch_row)


def kernel(x, edge_index, batch, W1l, W1r, b1, W2l, W2r, b2):
    x = x.astype(F32)
    src = edge_index[0].astype(jnp.int32)
    dst = edge_index[1].astype(jnp.int32)
    npad_e = E0 + E1 - E
    pad_src = jnp.zeros((npad_e,), jnp.int32)
    pad_dst = N + (jnp.arange(npad_e, dtype=jnp.int32) % (NPAD - N))
    srcp0 = src[:E0].reshape(NS, CPW0, CH)
    dstp0 = dst[:E0].reshape(NS, CPW0, CH)
    srcp1 = jnp.concatenate([src[E0:], pad_src]).reshape(NS, CPW1, CH)
    dstp1 = jnp.concatenate([dst[E0:], pad_dst]).reshape(NS, CPW1, CH)
    ztail = ((0, 0), (0, CPW0 - CPW1), (0, 0))
    srcp = jnp.concatenate([srcp0, jnp.pad(srcp1, ztail)], axis=0)
    dstp = jnp.concatenate([dstp0, jnp.pad(dstp1, ztail)], axis=0)
    zacc = jnp.zeros((NPAD, D), F32)
    zdeg = jnp.zeros((NPAD,), F32)
    b1r = b1.reshape(1, D)
    b2r = b2.reshape(1, D)
    batch_row = batch.astype(jnp.int32).reshape(1, N)

    y1, p1 = _tc_pre(x, W1l, W1r, b1r)
    sacc1, dega = _segsum_deg(y1, srcp, dstp, zacc, zdeg)
    degT = dega.T[:N]
    y2, p2 = _tc_mid(sacc1[0, :N], sacc1[1, :N], degT, p1, W2l, W2r, b2r)
    sacc2 = _segsum(y2, srcp, dstp, zacc)
    out = _tc_final(sacc2[0, :N], sacc2[1, :N], degT, p2, batch_row)
    return out
